# baseline (speedup 1.0000x reference)
import jax
import jax.numpy as jnp
from jax import lax
from jax.experimental import pallas as pl
from jax.experimental.pallas import tpu as pltpu
from jax.experimental.pallas import tpu_sc as plsc

NUM_CORES = 2
NUM_SUBCORES = 16
LANES = 16
NW = NUM_CORES * NUM_SUBCORES

BATCH = 16384
DIM = 32
BPW = BATCH // NW
CHUNKS = BPW // LANES


def _bpr_body(uid_hbm, pid_hbm, nid_hbm, uembT_hbm, iembT_hbm,
              outp_hbm, outn_hbm,
              uidx_v, pidx_v, nidx_v,
              ut_v, pt_v, nt_v,
              outp_v, outn_v, sems):
    wid = lax.axis_index("s") * NUM_CORES + lax.axis_index("c")
    base = wid * BPW

    pltpu.sync_copy(uid_hbm.at[pl.ds(base, BPW)], uidx_v)
    pltpu.sync_copy(pid_hbm.at[pl.ds(base, BPW)], pidx_v)
    pltpu.sync_copy(nid_hbm.at[pl.ds(base, BPW)], nidx_v)

    copies = []
    for d in range(DIM):
        s = pl.ds(d * BPW, BPW)
        copies.append(pltpu.async_copy(
            uembT_hbm.at[d].at[uidx_v], ut_v.at[s], sems.at[0]))
        copies.append(pltpu.async_copy(
            iembT_hbm.at[d].at[pidx_v], pt_v.at[s], sems.at[1]))
        copies.append(pltpu.async_copy(
            iembT_hbm.at[d].at[nidx_v], nt_v.at[s], sems.at[2]))
    for cp in copies:
        cp.wait()

    for c in range(CHUNKS):
        cb = c * LANES
        accp = jnp.zeros((LANES,), jnp.float32)
        accn = jnp.zeros((LANES,), jnp.float32)
        for d in range(DIM):
            s = pl.ds(d * BPW + cb, LANES)
            u = ut_v[s]
            accp = accp + u * pt_v[s]
            accn = accn + u * nt_v[s]
        o = pl.ds(cb, LANES)
        outp_v[o] = accp
        outn_v[o] = accn

    pltpu.sync_copy(outp_v, outp_hbm.at[pl.ds(base, BPW)])
    pltpu.sync_copy(outn_v, outn_hbm.at[pl.ds(base, BPW)])


def kernel(user_ids, pos_item_ids, neg_item_ids, user_emb, item_emb):
    mesh = plsc.VectorSubcoreMesh(
        core_axis_name="c", subcore_axis_name="s",
        num_cores=NUM_CORES, num_subcores=NUM_SUBCORES)
    out_type = (jax.ShapeDtypeStruct((BATCH,), jnp.float32),
                jax.ShapeDtypeStruct((BATCH,), jnp.float32))
    scratch = [
        pltpu.VMEM((BPW,), jnp.int32),
        pltpu.VMEM((BPW,), jnp.int32),
        pltpu.VMEM((BPW,), jnp.int32),
        pltpu.VMEM((DIM * BPW,), jnp.float32),
        pltpu.VMEM((DIM * BPW,), jnp.float32),
        pltpu.VMEM((DIM * BPW,), jnp.float32),
        pltpu.VMEM((BPW,), jnp.float32),
        pltpu.VMEM((BPW,), jnp.float32),
        pltpu.SemaphoreType.DMA((3,)),
    ]
    f = pl.kernel(_bpr_body, out_type=out_type, mesh=mesh,
                  scratch_types=scratch,
                  compiler_params=pltpu.CompilerParams(
                      needs_layout_passes=False,
                      use_tc_tiling_on_sc=False))
    return f(user_ids.astype(jnp.int32), pos_item_ids.astype(jnp.int32),
             neg_item_ids.astype(jnp.int32), user_emb.T, item_emb.T)


# speedup vs baseline: 5.6473x; 5.6473x over previous
import jax
import jax.numpy as jnp
from jax import lax
from jax.experimental import pallas as pl
from jax.experimental.pallas import tpu as pltpu
from jax.experimental.pallas import tpu_sc as plsc

NUM_CORES = 2
NUM_SUBCORES = 16
LANES = 16
NW = NUM_CORES * NUM_SUBCORES

BATCH = 16384
DIM = 32
BPW = BATCH // NW
CHUNKS = BPW // LANES


def _bpr_body(uid_hbm, pid_hbm, nid_hbm, uemb_hbm, iemb_hbm,
              outp_hbm, outn_hbm,
              uidx_v, pidx_v, nidx_v, urows_v, prows_v, nrows_v,
              outp_v, outn_v, sem_u, sem_p, sem_n):
    wid = lax.axis_index("s") * NUM_CORES + lax.axis_index("c")
    base = wid * BPW

    pltpu.sync_copy(uid_hbm.at[pl.ds(base, BPW)], uidx_v)
    pltpu.sync_copy(pid_hbm.at[pl.ds(base, BPW)], pidx_v)
    pltpu.sync_copy(nid_hbm.at[pl.ds(base, BPW)], nidx_v)

    cu = pltpu.async_copy(uemb_hbm.at[uidx_v], urows_v, sem_u)
    cp = pltpu.async_copy(iemb_hbm.at[pidx_v], prows_v, sem_p)
    cn = pltpu.async_copy(iemb_hbm.at[nidx_v], nrows_v, sem_n)
    cu.wait()
    cp.wait()
    cn.wait()

    def chunk(c, carry):
        rows = c * LANES + lax.iota(jnp.int32, LANES)
        accp = jnp.zeros((LANES,), jnp.float32)
        accn = jnp.zeros((LANES,), jnp.float32)
        for d in range(DIM):
            dv = jnp.full((LANES,), d, jnp.int32)
            u = plsc.load_gather(urows_v, [rows, dv])
            p = plsc.load_gather(prows_v, [rows, dv])
            n = plsc.load_gather(nrows_v, [rows, dv])
            accp = accp + u * p
            accn = accn + u * n
        outp_v[pl.ds(c * LANES, LANES)] = accp
        outn_v[pl.ds(c * LANES, LANES)] = accn
        return carry

    lax.fori_loop(0, CHUNKS, chunk, 0)

    pltpu.sync_copy(outp_v, outp_hbm.at[pl.ds(base, BPW)])
    pltpu.sync_copy(outn_v, outn_hbm.at[pl.ds(base, BPW)])


def kernel(user_ids, pos_item_ids, neg_item_ids, user_emb, item_emb):
    mesh = plsc.VectorSubcoreMesh(
        core_axis_name="c", subcore_axis_name="s",
        num_cores=NUM_CORES, num_subcores=NUM_SUBCORES)
    out_type = (jax.ShapeDtypeStruct((BATCH,), jnp.float32),
                jax.ShapeDtypeStruct((BATCH,), jnp.float32))
    scratch = [
        pltpu.VMEM((BPW,), jnp.int32),
        pltpu.VMEM((BPW,), jnp.int32),
        pltpu.VMEM((BPW,), jnp.int32),
        pltpu.VMEM((BPW, DIM), jnp.float32),
        pltpu.VMEM((BPW, DIM), jnp.float32),
        pltpu.VMEM((BPW, DIM), jnp.float32),
        pltpu.VMEM((BPW,), jnp.float32),
        pltpu.VMEM((BPW,), jnp.float32),
        pltpu.SemaphoreType.DMA,
        pltpu.SemaphoreType.DMA,
        pltpu.SemaphoreType.DMA,
    ]
    f = pl.kernel(_bpr_body, out_type=out_type, mesh=mesh,
                  scratch_types=scratch,
                  compiler_params=pltpu.CompilerParams(
                      needs_layout_passes=False,
                      use_tc_tiling_on_sc=False))
    return f(user_ids.astype(jnp.int32), pos_item_ids.astype(jnp.int32),
             neg_item_ids.astype(jnp.int32), user_emb, item_emb)


# speedup vs baseline: 9.0004x; 1.5937x over previous
import jax
import jax.numpy as jnp
from jax import lax
from jax.experimental import pallas as pl
from jax.experimental.pallas import tpu as pltpu
from jax.experimental.pallas import tpu_sc as plsc

NUM_CORES = 2
NUM_SUBCORES = 16
LANES = 16
NW = NUM_CORES * NUM_SUBCORES

BATCH = 16384
DIM = 32
PACK = 128 // DIM
BPW = BATCH // NW
RING = 8
ROUNDS = BPW // RING
QUARTER = 128
NQ = BPW // QUARTER
QCHUNKS = QUARTER // LANES


def _user_gather_body(uid_hbm, uembT_hbm, ucols_hbm,
                      uids_sm, uids_v, ring_v, ucols_v, sems):
    wid = lax.axis_index("s") * NUM_CORES + lax.axis_index("c")
    base = pl.multiple_of(wid * BPW, 128)

    pltpu.sync_copy(uid_hbm.at[pl.ds(base, BPW)], uids_v)

    lanes = lax.iota(jnp.int32, LANES)

    def load_ids(k, carry):
        vec = uids_v[pl.ds(k * LANES, LANES)]
        for l in range(LANES):
            b = lax.reduce_max(jnp.where(lanes == l, vec, 0), (0,))
            uids_sm[k * LANES + l] = b
        return carry

    lax.fori_loop(0, BPW // LANES, load_ids, 0)

    def enqueue(i, r):
        b = uids_sm[i]
        blk = pl.multiple_of(b & ~(128 - 1), 128)
        pltpu.async_copy(uembT_hbm.at[:, pl.ds(blk, 128)],
                         ring_v.at[r], sems.at[r])

    def extract(i, r):
        pltpu.make_async_copy(uembT_hbm.at[:, pl.ds(0, 128)],
                              ring_v.at[r], sems.at[r]).wait()
        j = jnp.full((LANES,), uids_sm[i] & (128 - 1), jnp.int32)
        rows = lax.iota(jnp.int32, LANES)
        col = jnp.full((LANES,), i, jnp.int32)
        u0 = plsc.load_gather(ring_v.at[r], [rows, j])
        u1 = plsc.load_gather(ring_v.at[r], [rows + LANES, j])
        plsc.store_scatter(ucols_v, [rows, col], u0)
        plsc.store_scatter(ucols_v, [rows + LANES, col], u1)

    for r in range(RING):
        enqueue(r, r)

    def round_body(k, carry):
        i0 = k * RING
        for r in range(RING):
            extract(i0 + r, r)
            enqueue(i0 + RING + r, r)
        return carry

    lax.fori_loop(0, ROUNDS - 1, round_body, 0)
    for r in range(RING):
        extract((ROUNDS - 1) * RING + r, r)

    pltpu.sync_copy(ucols_v, ucols_hbm.at[:, pl.ds(base, BPW)])


def _score_body(pid_hbm, nid_hbm, iemb_hbm, ucols_hbm,
                outp_hbm, outn_hbm,
                pidx_v, nidx_v, pgidx_v, ngidx_v,
                ucols_v, prows_v, nrows_v,
                outp_v, outn_v, sems):
    wid = lax.axis_index("s") * NUM_CORES + lax.axis_index("c")
    base = pl.multiple_of(wid * BPW, 128)

    pltpu.sync_copy(pid_hbm.at[pl.ds(base, BPW)], pidx_v)
    pltpu.sync_copy(nid_hbm.at[pl.ds(base, BPW)], nidx_v)
    pltpu.sync_copy(ucols_hbm.at[:, pl.ds(base, BPW)], ucols_v)

    for jj in range(BPW // LANES):
        s = pl.ds(jj * LANES, LANES)
        pgidx_v[s] = jax.lax.shift_right_logical(pidx_v[s], 2)
        ngidx_v[s] = jax.lax.shift_right_logical(nidx_v[s], 2)

    def fire(q, buf):
        s = pl.ds(q * QUARTER, QUARTER)
        return (
            pltpu.async_copy(iemb_hbm.at[pgidx_v.at[s]], prows_v.at[buf],
                             sems.at[buf, 0]),
            pltpu.async_copy(iemb_hbm.at[ngidx_v.at[s]], nrows_v.at[buf],
                             sems.at[buf, 1]),
        )

    def compute(q, buf):
        qb = q * QUARTER
        for c in range(QCHUNKS):
            rows = c * LANES + lax.iota(jnp.int32, LANES)
            s = pl.ds(qb + c * LANES, LANES)
            ucol = qb + rows
            poff = (pidx_v[s] & (PACK - 1)) * DIM
            noff = (nidx_v[s] & (PACK - 1)) * DIM
            accp = jnp.zeros((LANES,), jnp.float32)
            accn = jnp.zeros((LANES,), jnp.float32)
            for d in range(DIM):
                dv = jnp.full((LANES,), d, jnp.int32)
                u = plsc.load_gather(ucols_v, [dv, ucol])
                p = plsc.load_gather(prows_v.at[buf], [rows, poff + d])
                n = plsc.load_gather(nrows_v.at[buf], [rows, noff + d])
                accp = accp + u * p
                accn = accn + u * n
            outp_v[s] = accp
            outn_v[s] = accn

    copies = fire(0, 0)
    for q in range(NQ):
        nxt = None
        if q + 1 < NQ:
            nxt = fire(q + 1, (q + 1) % 2)
        for cp in copies:
            cp.wait()
        compute(q, q % 2)
        copies = nxt

    pltpu.sync_copy(outp_v, outp_hbm.at[pl.ds(base, BPW)])
    pltpu.sync_copy(outn_v, outn_hbm.at[pl.ds(base, BPW)])


def kernel(user_ids, pos_item_ids, neg_item_ids, user_emb, item_emb):
    mesh = plsc.VectorSubcoreMesh(
        core_axis_name="c", subcore_axis_name="s",
        num_cores=NUM_CORES, num_subcores=NUM_SUBCORES)

    k1 = pl.kernel(
        _user_gather_body,
        out_type=jax.ShapeDtypeStruct((DIM, BATCH), jnp.float32),
        mesh=mesh,
        scratch_types=[
            pltpu.SMEM((BPW,), jnp.int32),
            pltpu.VMEM((BPW,), jnp.int32),
            pltpu.VMEM((RING, DIM, 128), jnp.float32),
            pltpu.VMEM((DIM, BPW), jnp.float32),
            pltpu.SemaphoreType.DMA((RING,)),
        ],
        compiler_params=pltpu.CompilerParams(
            needs_layout_passes=False,
            use_tc_tiling_on_sc=True))

    k2 = pl.kernel(
        _score_body,
        out_type=(jax.ShapeDtypeStruct((BATCH,), jnp.float32),
                  jax.ShapeDtypeStruct((BATCH,), jnp.float32)),
        mesh=mesh,
        scratch_types=[
            pltpu.VMEM((BPW,), jnp.int32),
            pltpu.VMEM((BPW,), jnp.int32),
            pltpu.VMEM((BPW,), jnp.int32),
            pltpu.VMEM((BPW,), jnp.int32),
            pltpu.VMEM((DIM, BPW), jnp.float32),
            pltpu.VMEM((2, QUARTER, 128), jnp.float32),
            pltpu.VMEM((2, QUARTER, 128), jnp.float32),
            pltpu.VMEM((BPW,), jnp.float32),
            pltpu.VMEM((BPW,), jnp.float32),
            pltpu.SemaphoreType.DMA((2, 2)),
        ],
        compiler_params=pltpu.CompilerParams(
            needs_layout_passes=False,
            use_tc_tiling_on_sc=True))

    ni = item_emb.shape[0]
    iemb2 = item_emb.reshape(ni // PACK, DIM * PACK)
    ucols = k1(user_ids.astype(jnp.int32), user_emb.T)
    return k2(pos_item_ids.astype(jnp.int32),
              neg_item_ids.astype(jnp.int32), iemb2, ucols)


# speedup vs baseline: 15.4092x; 1.7121x over previous
import jax
import jax.numpy as jnp
from jax import lax
from jax.experimental import pallas as pl
from jax.experimental.pallas import tpu as pltpu
from jax.experimental.pallas import tpu_sc as plsc

NUM_CORES = 2
NUM_SUBCORES = 16
LANES = 16
NW = NUM_CORES * NUM_SUBCORES

BATCH = 16384
DIM = 32
BPW = BATCH // NW
RING = 8
ROUNDS = BPW // RING
CHUNKS = BPW // LANES


def _bpr_body(uid_hbm, pid_hbm, nid_hbm, uembT_hbm, iembT_hbm,
              outp_hbm, outn_hbm,
              ids_sm, ids_v, ring_v, ucols_v, pcols_v, ncols_v,
              outp_v, outn_v, sems):
    wid = lax.axis_index("s") * NUM_CORES + lax.axis_index("c")
    base = pl.multiple_of(wid * BPW, 128)
    lanes = lax.iota(jnp.int32, LANES)

    def gather_role(id_hbm, tab_hbm, cols_v):
        pltpu.sync_copy(id_hbm.at[pl.ds(base, BPW)], ids_v)

        def load_ids(k, carry):
            vec = ids_v[pl.ds(k * LANES, LANES)]
            for l in range(LANES):
                b = lax.reduce_max(jnp.where(lanes == l, vec, 0), (0,))
                ids_sm[k * LANES + l] = b
            return carry

        lax.fori_loop(0, BPW // LANES, load_ids, 0)

        def enqueue(i, r):
            blk = pl.multiple_of(ids_sm[i] & ~(128 - 1), 128)
            pltpu.async_copy(tab_hbm.at[:, pl.ds(blk, 128)],
                             ring_v.at[r], sems.at[r])

        def extract(i, r):
            pltpu.make_async_copy(tab_hbm.at[:, pl.ds(0, 128)],
                                  ring_v.at[r], sems.at[r]).wait()
            j = jnp.full((LANES,), ids_sm[i] & (128 - 1), jnp.int32)
            col = jnp.full((LANES,), i, jnp.int32)
            v0 = plsc.load_gather(ring_v.at[r], [lanes, j])
            v1 = plsc.load_gather(ring_v.at[r], [lanes + LANES, j])
            plsc.store_scatter(cols_v, [lanes, col], v0)
            plsc.store_scatter(cols_v, [lanes + LANES, col], v1)

        for r in range(RING):
            enqueue(r, r)

        def round_body(k, carry):
            i0 = k * RING
            for r in range(RING):
                extract(i0 + r, r)
                enqueue(i0 + RING + r, r)
            return carry

        lax.fori_loop(0, ROUNDS - 1, round_body, 0)
        for r in range(RING):
            extract((ROUNDS - 1) * RING + r, r)

    gather_role(uid_hbm, uembT_hbm, ucols_v)
    gather_role(pid_hbm, iembT_hbm, pcols_v)
    gather_role(nid_hbm, iembT_hbm, ncols_v)

    def chunk(c, carry):
        col = c * LANES + lanes
        accp = jnp.zeros((LANES,), jnp.float32)
        accn = jnp.zeros((LANES,), jnp.float32)
        for d in range(DIM):
            dv = jnp.full((LANES,), d, jnp.int32)
            u = plsc.load_gather(ucols_v, [dv, col])
            p = plsc.load_gather(pcols_v, [dv, col])
            n = plsc.load_gather(ncols_v, [dv, col])
            accp = accp + u * p
            accn = accn + u * n
        outp_v[pl.ds(c * LANES, LANES)] = accp
        outn_v[pl.ds(c * LANES, LANES)] = accn
        return carry

    lax.fori_loop(0, CHUNKS, chunk, 0)

    pltpu.sync_copy(outp_v, outp_hbm.at[pl.ds(base, BPW)])
    pltpu.sync_copy(outn_v, outn_hbm.at[pl.ds(base, BPW)])


def kernel(user_ids, pos_item_ids, neg_item_ids, user_emb, item_emb):
    mesh = plsc.VectorSubcoreMesh(
        core_axis_name="c", subcore_axis_name="s",
        num_cores=NUM_CORES, num_subcores=NUM_SUBCORES)
    out_type = (jax.ShapeDtypeStruct((BATCH,), jnp.float32),
                jax.ShapeDtypeStruct((BATCH,), jnp.float32))
    scratch = [
        pltpu.SMEM((BPW,), jnp.int32),
        pltpu.VMEM((BPW,), jnp.int32),
        pltpu.VMEM((RING, DIM, 128), jnp.float32),
        pltpu.VMEM((DIM, BPW), jnp.float32),
        pltpu.VMEM((DIM, BPW), jnp.float32),
        pltpu.VMEM((DIM, BPW), jnp.float32),
        pltpu.VMEM((BPW,), jnp.float32),
        pltpu.VMEM((BPW,), jnp.float32),
        pltpu.SemaphoreType.DMA((RING,)),
    ]
    f = pl.kernel(_bpr_body, out_type=out_type, mesh=mesh,
                  scratch_types=scratch,
                  compiler_params=pltpu.CompilerParams(
                      needs_layout_passes=False,
                      use_tc_tiling_on_sc=True))
    return f(user_ids.astype(jnp.int32), pos_item_ids.astype(jnp.int32),
             neg_item_ids.astype(jnp.int32), user_emb.T, item_emb.T)
